# baseline (speedup 1.0000x reference)
import jax
import jax.numpy as jnp
from jax.experimental import pallas as pl
from jax.experimental.pallas import tpu as pltpu

_N = 4096
_D = 512
_BM = 512


def _support_body(h_ref, w_ref, sup_ref):
    hb = h_ref[...].astype(jnp.bfloat16)
    wb = w_ref[...].astype(jnp.bfloat16)
    sup = jnp.dot(hb, wb, preferred_element_type=jnp.float32)
    sup_ref[...] = sup.astype(jnp.bfloat16)


def _spmm_body(adj_ref, sup_ref, b_ref, out_ref):
    ab = adj_ref[...].astype(jnp.bfloat16)
    acc = jnp.dot(ab, sup_ref[...], preferred_element_type=jnp.float32)
    out_ref[...] = jnp.maximum(acc + b_ref[...], 0.0)


def kernel(h, adj, W, b):
    b2 = b.reshape(1, _D)
    sup = pl.pallas_call(
        _support_body,
        in_specs=[
            pl.BlockSpec((_N, _D), lambda: (0, 0)),
            pl.BlockSpec((_D, _D), lambda: (0, 0)),
        ],
        out_specs=pl.BlockSpec((_N, _D), lambda: (0, 0)),
        out_shape=jax.ShapeDtypeStruct((_N, _D), jnp.bfloat16),
    )(h, W)
    return pl.pallas_call(
        _spmm_body,
        grid=(_N // _BM,),
        in_specs=[
            pl.BlockSpec((_BM, _N), lambda i: (i, 0)),
            pl.BlockSpec((_N, _D), lambda i: (0, 0)),
            pl.BlockSpec((1, _D), lambda i: (0, 0)),
        ],
        out_specs=pl.BlockSpec((_BM, _D), lambda i: (i, 0)),
        out_shape=jax.ShapeDtypeStruct((_N, _D), jnp.float32),
        compiler_params=pltpu.CompilerParams(
            dimension_semantics=("parallel",),
        ),
    )(adj, sup, b2)


# speedup vs baseline: 1.0398x; 1.0398x over previous
import jax
import jax.numpy as jnp
from jax.experimental import pallas as pl
from jax.experimental.pallas import tpu as pltpu

_N = 4096
_D = 512
_BK = 512
_NB = _N // _BK


def _gcn_body(adj_ref, h_ref, w_ref, b_ref, out_ref):
    k = pl.program_id(0)

    supk = jnp.dot(h_ref[...].astype(jnp.bfloat16),
                   w_ref[...].astype(jnp.bfloat16),
                   preferred_element_type=jnp.float32).astype(jnp.bfloat16)
    part = jnp.dot(adj_ref[...].astype(jnp.bfloat16), supk,
                   preferred_element_type=jnp.float32)

    @pl.when(k == 0)
    def _init():
        out_ref[...] = part

    @pl.when((k > 0) & (k < _NB - 1))
    def _accum():
        out_ref[...] += part

    @pl.when(k == _NB - 1)
    def _final():
        out_ref[...] = jnp.maximum(out_ref[...] + part + b_ref[...], 0.0)


def kernel(h, adj, W, b):
    b2 = b.reshape(1, _D)
    return pl.pallas_call(
        _gcn_body,
        grid=(_NB,),
        in_specs=[
            pl.BlockSpec((_N, _BK), lambda k: (0, k)),
            pl.BlockSpec((_BK, _D), lambda k: (k, 0)),
            pl.BlockSpec((_D, _D), lambda k: (0, 0)),
            pl.BlockSpec((1, _D), lambda k: (0, 0)),
        ],
        out_specs=pl.BlockSpec((_N, _D), lambda k: (0, 0)),
        out_shape=jax.ShapeDtypeStruct((_N, _D), jnp.float32),
        compiler_params=pltpu.CompilerParams(
            dimension_semantics=("arbitrary",),
        ),
    )(adj, h, W, b2)


# speedup vs baseline: 1.0985x; 1.0565x over previous
import jax
import jax.numpy as jnp
from jax.experimental import pallas as pl
from jax.experimental.pallas import tpu as pltpu

_N = 4096
_D = 512
_BM = 512
_NB = _N // _BM


def _gcn_body(adj_ref, h_ref, w_ref, b_ref, out_ref, sup_ref):
    k = pl.program_id(0)

    @pl.when(k == 0)
    def _support():
        sup_ref[...] = jnp.dot(
            h_ref[...].astype(jnp.bfloat16),
            w_ref[...].astype(jnp.bfloat16),
            preferred_element_type=jnp.float32,
        ).astype(jnp.bfloat16)

    @pl.when(k > 0)
    def _block():
        acc = jnp.dot(
            adj_ref[...].astype(jnp.bfloat16),
            sup_ref[...],
            preferred_element_type=jnp.float32,
        )
        out_ref[...] = jnp.maximum(acc + b_ref[...], 0.0)


def kernel(h, adj, W, b):
    b2 = b.reshape(1, _D)
    return pl.pallas_call(
        _gcn_body,
        grid=(_NB + 1,),
        in_specs=[
            pl.BlockSpec((_BM, _N), lambda k: (jnp.maximum(k - 1, 0), 0)),
            pl.BlockSpec((_N, _D), lambda k: (0, 0)),
            pl.BlockSpec((_D, _D), lambda k: (0, 0)),
            pl.BlockSpec((1, _D), lambda k: (0, 0)),
        ],
        out_specs=pl.BlockSpec((_BM, _D), lambda k: (jnp.maximum(k - 1, 0), 0)),
        out_shape=jax.ShapeDtypeStruct((_N, _D), jnp.float32),
        scratch_shapes=[pltpu.VMEM((_N, _D), jnp.bfloat16)],
        compiler_params=pltpu.CompilerParams(
            dimension_semantics=("arbitrary",),
        ),
    )(adj, h, W, b2)
